# baseline (speedup 1.0000x reference)
import jax
import jax.numpy as jnp
from jax import lax
from jax.experimental import pallas as pl
from jax.experimental.pallas import tpu as pltpu
from jax.experimental.pallas import tpu_sc as plsc

B = 4
N = 10000
SEQ = 16
IN = 256
OUT = 256
M = 2500
NNZ = 20000

R = B * N
NW = 32
ROWS_PER_W = R // NW

_MM_RB = 1000


def _pack_bf16(res):
    u = lax.bitcast_convert_type(res, jnp.uint32)
    rnd = u + jnp.uint32(0x7FFF) + ((u >> 16) & jnp.uint32(1))
    lo = rnd[:, :OUT // 2] >> 16
    hi = rnd[:, OUT // 2:] & jnp.uint32(0xFFFF0000)
    return lax.bitcast_convert_type(lo | hi, jnp.int32)


def _mm_body(x_ref, w_ref, z_ref):
    res = lax.dot_general(
        x_ref[...].astype(jnp.bfloat16), w_ref[...], (((1,), (0,)), ((), ())),
        preferred_element_type=jnp.float32)
    z_ref[0] = _pack_bf16(res[:, :OUT])
    z_ref[1] = _pack_bf16(res[:, OUT:])


def _spiral_matmul(x2, w2):
    nib = R // _MM_RB
    return pl.pallas_call(
        _mm_body,
        grid=(nib, SEQ // 2),
        in_specs=[
            pl.BlockSpec((_MM_RB, IN), lambda i, q: (i, 0)),
            pl.BlockSpec((IN, 2 * OUT), lambda i, q: (q, 0)),
        ],
        out_specs=pl.BlockSpec((2, _MM_RB, OUT // 2), lambda i, q: (q, i, 0)),
        out_shape=jax.ShapeDtypeStruct((SEQ, R, OUT // 2), jnp.int32),
    )(x2, w2)


_GS_CH = 8
_GS_TR = 1248

_SC_MESH = plsc.VectorSubcoreMesh(core_axis_name="c", subcore_axis_name="s")


def _fused_body(z4_hbm, idx_hbm, bias_hbm, ents_hbm, bounds_hbm,
                pooled_hbm, out_hbm,
                idx8_a, idx8_b, gidx_a, gidx_b, grow_a, grow_b,
                outb_v, bias_v, ent_a, ent_b, acc_v, bounds_v,
                sem_a, sem_b, sem_ia, sem_ib):
    cid = lax.axis_index("c")
    sid = lax.axis_index("s")
    row_base = cid * (R // 2) + sid * _GS_TR
    nrows = jnp.where(sid == 15, R // 2 - 15 * _GS_TR, _GS_TR)
    nch = nrows // _GS_CH
    pltpu.sync_copy(bias_hbm, bias_v)
    iota16 = lax.broadcasted_iota(jnp.int32, (16,), 0)

    def idx_src(k):
        r0 = row_base + k * _GS_CH
        n0 = pl.multiple_of(r0 % N, _GS_CH)
        return idx_hbm.at[pl.ds(n0, _GS_CH)]

    def idx_start(k, idx8, sem):
        return pltpu.async_copy(idx_src(k), idx8, sem)

    def build_start(k, idx8, isem, gidx, grow, sem):
        pltpu.make_async_copy(idx_src(k), idx8, isem).wait()
        r0 = row_base + k * _GS_CH
        svec = iota16 * R + (r0 // N) * N
        for j in range(_GS_CH):
            gidx[pl.ds(j * SEQ, SEQ)] = idx8[j, :] + svec
        h = _GS_CH * SEQ // 2
        pltpu.async_copy(z4_hbm.at[gidx.at[pl.ds(0, h)]],
                         grow.at[pl.ds(0, h)], sem)
        pltpu.async_copy(z4_hbm.at[gidx.at[pl.ds(h, h)]],
                         grow.at[pl.ds(h, h)], sem)

    def wait_for(gidx, grow, sem):
        h = _GS_CH * SEQ // 2
        pltpu.make_async_copy(z4_hbm.at[gidx.at[pl.ds(0, h)]],
                              grow.at[pl.ds(0, h)], sem).wait()
        pltpu.make_async_copy(z4_hbm.at[gidx.at[pl.ds(h, h)]],
                              grow.at[pl.ds(h, h)], sem).wait()

    mask_hi = jnp.int32(-65536)

    def compute(k, grow):
        def node(j, _):
            for t in range(OUT // 32):
                cs = pl.ds(t * 16, 16)
                w0 = grow[j * SEQ, cs]
                acc_lo = lax.bitcast_convert_type(w0 << 16, jnp.float32)
                acc_hi = lax.bitcast_convert_type(w0, jnp.float32)
                for s in range(1, SEQ):
                    w = grow[j * SEQ + s, cs]
                    acc_lo = acc_lo + lax.bitcast_convert_type(w << 16, jnp.float32)
                    acc_hi = acc_hi + lax.bitcast_convert_type(w, jnp.float32)
                acc_lo = acc_lo + bias_v[cs]
                acc_hi = acc_hi + bias_v[pl.ds(OUT // 2 + t * 16, 16)]
                elu_lo = jnp.where(acc_lo > 0.0, acc_lo, jnp.exp(acc_lo) - 1.0)
                elu_hi = jnp.where(acc_hi > 0.0, acc_hi, jnp.exp(acc_hi) - 1.0)
                ulo = lax.bitcast_convert_type(elu_lo, jnp.uint32)
                ulo = (ulo + jnp.uint32(0x7FFF) + ((ulo >> 16) & jnp.uint32(1))) >> 16
                uhi = lax.bitcast_convert_type(elu_hi, jnp.uint32)
                uhi = (uhi + jnp.uint32(0x7FFF) + ((uhi >> 16) & jnp.uint32(1))) & jnp.uint32(0xFFFF0000)
                outb_v[j, cs] = lax.bitcast_convert_type(ulo | uhi, jnp.int32)
            return 0
        lax.fori_loop(0, _GS_CH, node, 0)
        r0 = pl.multiple_of(row_base + k * _GS_CH, _GS_CH)
        pltpu.sync_copy(outb_v, out_hbm.at[pl.ds(r0, _GS_CH)])

    idx_start(0, idx8_a, sem_ia)
    build_start(0, idx8_a, sem_ia, gidx_a, grow_a, sem_a)
    idx_start(1, idx8_b, sem_ib)

    def pair(k2, _):
        k = 2 * k2
        build_start(k + 1, idx8_b, sem_ib, gidx_b, grow_b, sem_b)
        idx_start(k + 2, idx8_a, sem_ia)
        wait_for(gidx_a, grow_a, sem_a)
        compute(k, grow_a)

        pltpu.make_async_copy(idx_src(k + 2), idx8_a, sem_ia).wait()

        @pl.when(k + 2 < nch)
        def _():
            svec = iota16 * R + ((row_base + (k + 2) * _GS_CH) // N) * N
            for j in range(_GS_CH):
                gidx_a[pl.ds(j * SEQ, SEQ)] = idx8_a[j, :] + svec
            h = _GS_CH * SEQ // 2
            pltpu.async_copy(z4_hbm.at[gidx_a.at[pl.ds(0, h)]],
                             grow_a.at[pl.ds(0, h)], sem_a)
            pltpu.async_copy(z4_hbm.at[gidx_a.at[pl.ds(h, h)]],
                             grow_a.at[pl.ds(h, h)], sem_a)
        idx_start(k + 3, idx8_b, sem_ib)
        wait_for(gidx_b, grow_b, sem_b)
        compute(k + 1, grow_b)
        return 0

    lax.fori_loop(0, nch // 2, pair, 0)
    pltpu.make_async_copy(idx_src(0), idx8_b, sem_ib).wait()

    plsc.subcore_barrier()

    b = 2 * cid + sid // 8
    t8 = sid % 8
    r_lo = t8 * _RPT

    def zrow(i, _):
        for t in range(OUT // 16):
            acc_v[pl.ds(i * OUT + t * 16, 16)] = jnp.zeros((16,), jnp.float32)
        return 0
    lax.fori_loop(0, _RPT_LAST, zrow, 0)

    wid = cid * 16 + sid
    pltpu.sync_copy(bounds_hbm.at[pl.ds(wid * 16, 16)], bounds_v)
    bvec = bounds_v[pl.ds(0, 16)]
    ent_lo = bvec[0]
    ent_hi = bvec[1]
    cblk0 = ent_lo // _P_C
    nch3 = (ent_hi - cblk0 * _P_C + _P_C - 1) // _P_C

    def fetch_start3(c, ent_v, gidx_v, gbuf_v, sem):
        eoff = pl.multiple_of((cblk0 + c) * (3 * _P_C), 3 * _P_C)
        pltpu.sync_copy(ents_hbm.at[pl.ds(eoff, 3 * _P_C)], ent_v)
        for k in range(_P_C // 16):
            ks = pl.ds(k * 16, 16)
            gidx_v[ks] = ent_v[pl.ds(_P_C + k * 16, 16)] + b * N
        return pltpu.async_copy(out_hbm.at[gidx_v], gbuf_v, sem)

    def process3(c, ent_v, gidx_v, gbuf_v, sem):
        pltpu.make_async_copy(out_hbm.at[gidx_v], gbuf_v, sem).wait()
        e_base = (cblk0 + c) * _P_C

        def group_body(g, _):
            gs = pl.ds(g * 16, 16)
            eid = e_base + g * 16 + iota16
            valid = (eid >= ent_lo) & (eid < ent_hi)
            vals16 = lax.bitcast_convert_type(
                ent_v[pl.ds(2 * _P_C + g * 16, 16)], jnp.float32)
            bcv = jnp.where(valid, vals16, 0.0)
            rowsv = ent_v[gs]
            for j in range(16):
                bc = _lane_bcast(bcv, j)
                lrow = jnp.clip(rowsv[j] - r_lo, 0, _RPT_LAST - 1)
                e = g * 16 + j
                for t in range(OUT // 32):
                    w = gbuf_v[e, pl.ds(t * 16, 16)]
                    lo = lax.bitcast_convert_type(w << 16, jnp.float32)
                    hi = lax.bitcast_convert_type(w & mask_hi, jnp.float32)
                    cl = pl.ds(lrow * OUT + t * 16, 16)
                    ch = pl.ds(lrow * OUT + OUT // 2 + t * 16, 16)
                    acc_v[cl] = acc_v[cl] + lo * bc
                    acc_v[ch] = acc_v[ch] + hi * bc
            return 0
        lax.fori_loop(0, _P_C // 16, group_body, 0)

    @pl.when(nch3 > 0)
    def _():
        fetch_start3(0, ent_a, gidx_a, grow_a, sem_a)

    def pairs3(k2, _):
        k = 2 * k2

        @pl.when(k + 1 < nch3)
        def _():
            fetch_start3(k + 1, ent_b, gidx_b, grow_b, sem_b)
        process3(k, ent_a, gidx_a, grow_a, sem_a)

        @pl.when(k + 2 < nch3)
        def _():
            fetch_start3(k + 2, ent_a, gidx_a, grow_a, sem_a)

        @pl.when(k + 1 < nch3)
        def _():
            process3(k + 1, ent_b, gidx_b, grow_b, sem_b)
        return 0

    lax.fori_loop(0, (nch3 + 1) // 2, pairs3, 0)

    @pl.when(t8 < 7)
    def _():
        pltpu.sync_copy(
            acc_v.at[pl.ds(0, _RPT * OUT)],
            pooled_hbm.at[pl.ds(b * (M * OUT) + r_lo * OUT, _RPT * OUT)])

    @pl.when(t8 == 7)
    def _():
        rows_last = M - 7 * _RPT
        pltpu.sync_copy(
            acc_v.at[pl.ds(0, rows_last * OUT)],
            pooled_hbm.at[pl.ds(b * (M * OUT) + 7 * _RPT * OUT, rows_last * OUT)])


NNZ_PAD = 20480
_P_C = 128
_NEB = NNZ_PAD // _P_C
M_PAD = 2504
_RPT = 312
_RPT_LAST = M_PAD - 7 * _RPT
_BCAST_DNUMS = lax.GatherDimensionNumbers(
    offset_dims=(), collapsed_slice_dims=(0,), start_index_map=(0,))


def _lane_bcast(v, j):
    return lax.gather(v, jnp.full((16, 1), j, jnp.int32), _BCAST_DNUMS,
                      (1,), mode=lax.GatherScatterMode.PROMISE_IN_BOUNDS)


def _fused(z4, spiral_idx, bias, ents, bounds):
    f = pl.kernel(
        _fused_body,
        out_type=(jax.ShapeDtypeStruct((B * M * OUT,), jnp.float32),
                  jax.ShapeDtypeStruct((R, OUT // 2), jnp.int32)),
        mesh=_SC_MESH,
        scratch_types=[
            pltpu.VMEM((_GS_CH, SEQ), jnp.int32),
            pltpu.VMEM((_GS_CH, SEQ), jnp.int32),
            pltpu.VMEM((_GS_CH * SEQ,), jnp.int32),
            pltpu.VMEM((_GS_CH * SEQ,), jnp.int32),
            pltpu.VMEM((_GS_CH * SEQ, OUT // 2), jnp.int32),
            pltpu.VMEM((_GS_CH * SEQ, OUT // 2), jnp.int32),
            pltpu.VMEM((_GS_CH, OUT // 2), jnp.int32),
            pltpu.VMEM((OUT,), jnp.float32),
            pltpu.VMEM((3 * _P_C,), jnp.int32),
            pltpu.VMEM((3 * _P_C,), jnp.int32),
            pltpu.VMEM((_RPT_LAST * OUT,), jnp.float32),
            pltpu.VMEM((16,), jnp.int32),
            pltpu.SemaphoreType.DMA,
            pltpu.SemaphoreType.DMA,
            pltpu.SemaphoreType.DMA,
            pltpu.SemaphoreType.DMA,
        ],
    )
    pooled, _ = f(z4, spiral_idx, bias, ents, bounds)
    return pooled


def kernel(x, spiral_indices, trans_row, trans_col, trans_val, W, b):
    x2 = x.reshape(R, IN)
    w2 = (W.astype(jnp.bfloat16)
          .reshape(SEQ // 2, 2, IN, OUT).transpose(0, 2, 1, 3)
          .reshape((SEQ // 2) * IN, 2 * OUT))
    z4 = _spiral_matmul(x2, w2).reshape(SEQ * R, OUT // 2)

    pad = NNZ_PAD - NNZ
    rows32 = trans_row.astype(jnp.int32)
    rows_p = jnp.concatenate([rows32, jnp.zeros((pad,), jnp.int32)])
    cols_p = jnp.concatenate(
        [trans_col.astype(jnp.int32), jnp.zeros((pad,), jnp.int32)])
    vals_p = jnp.concatenate([trans_val, jnp.zeros((pad,), jnp.float32)])
    ents = jnp.concatenate(
        [rows_p.reshape(_NEB, _P_C), cols_p.reshape(_NEB, _P_C),
         lax.bitcast_convert_type(vals_p, jnp.int32).reshape(_NEB, _P_C)],
        axis=1).reshape(_NEB * 3 * _P_C)
    starts = jnp.arange(8, dtype=jnp.int32) * _RPT
    ss = jnp.searchsorted(rows32, starts, side="left").astype(jnp.int32)
    ends = jnp.concatenate([ss[1:], jnp.array([NNZ], jnp.int32)])
    pair16 = jnp.pad(jnp.stack([ss, ends], axis=1), ((0, 0), (0, 14)))
    bounds = jnp.tile(pair16, (4, 1)).reshape(32 * 16)
    pooled = _fused(z4, spiral_indices.astype(jnp.int32), b, ents, bounds)
    return pooled.reshape(B, M, OUT)


# speedup vs baseline: 1.0213x; 1.0213x over previous
import jax
import jax.numpy as jnp
from jax import lax
from jax.experimental import pallas as pl
from jax.experimental.pallas import tpu as pltpu
from jax.experimental.pallas import tpu_sc as plsc

B = 4
N = 10000
SEQ = 16
IN = 256
OUT = 256
M = 2500
NNZ = 20000

R = B * N
NW = 32
ROWS_PER_W = R // NW

_MM_RB = 1000


def _pack_bf16(res):
    u = lax.bitcast_convert_type(res, jnp.uint32)
    rnd = u + jnp.uint32(0x7FFF) + ((u >> 16) & jnp.uint32(1))
    lo = rnd[:, :OUT // 2] >> 16
    hi = rnd[:, OUT // 2:] & jnp.uint32(0xFFFF0000)
    return lax.bitcast_convert_type(lo | hi, jnp.int32)


def _mm_body(x_ref, w_ref, z_ref):
    res = lax.dot_general(
        x_ref[...].astype(jnp.bfloat16), w_ref[...], (((1,), (0,)), ((), ())),
        preferred_element_type=jnp.float32)
    z_ref[0] = _pack_bf16(res[:, :OUT])
    z_ref[1] = _pack_bf16(res[:, OUT:])


def _spiral_matmul(x2, w2):
    nib = R // _MM_RB
    return pl.pallas_call(
        _mm_body,
        grid=(nib, SEQ // 2),
        in_specs=[
            pl.BlockSpec((_MM_RB, IN), lambda i, q: (i, 0)),
            pl.BlockSpec((IN, 2 * OUT), lambda i, q: (q, 0)),
        ],
        out_specs=pl.BlockSpec((2, _MM_RB, OUT // 2), lambda i, q: (q, i, 0)),
        out_shape=jax.ShapeDtypeStruct((SEQ, R, OUT // 2), jnp.int32),
    )(x2, w2)


_GS_CH = 16
_GS_TR = 1248

_SC_MESH = plsc.VectorSubcoreMesh(core_axis_name="c", subcore_axis_name="s")


def _gather_sum_body(z4_hbm, idx_hbm, bias_hbm, out_hbm,
                     idx8_a, idx8_b, gidx_a, gidx_b, grow_a, grow_b,
                     outb_v, bias_v, sem_a, sem_b, sem_ia, sem_ib):
    wid = lax.axis_index("c") * 16 + lax.axis_index("s")
    row_base = wid * _GS_TR
    nrows = jnp.where(wid == 31, R - 31 * _GS_TR, _GS_TR)
    nch = nrows // _GS_CH
    pltpu.sync_copy(bias_hbm, bias_v)
    iota16 = lax.broadcasted_iota(jnp.int32, (16,), 0)

    def idx_src(k):
        r0 = row_base + k * _GS_CH
        n0 = pl.multiple_of(r0 % N, _GS_CH)
        return idx_hbm.at[pl.ds(n0, _GS_CH)]

    def idx_start(k, idx8, sem):
        return pltpu.async_copy(idx_src(k), idx8, sem)

    def build_start(k, idx8, isem, gidx, grow, sem):
        pltpu.make_async_copy(idx_src(k), idx8, isem).wait()
        r0 = row_base + k * _GS_CH
        svec = iota16 * R + (r0 // N) * N
        for j in range(_GS_CH):
            gidx[pl.ds(j * SEQ, SEQ)] = idx8[j, :] + svec
        h = _GS_CH * SEQ // 2
        pltpu.async_copy(z4_hbm.at[gidx.at[pl.ds(0, h)]],
                         grow.at[pl.ds(0, h)], sem)
        pltpu.async_copy(z4_hbm.at[gidx.at[pl.ds(h, h)]],
                         grow.at[pl.ds(h, h)], sem)

    def wait_for(gidx, grow, sem):
        h = _GS_CH * SEQ // 2
        pltpu.make_async_copy(z4_hbm.at[gidx.at[pl.ds(0, h)]],
                              grow.at[pl.ds(0, h)], sem).wait()
        pltpu.make_async_copy(z4_hbm.at[gidx.at[pl.ds(h, h)]],
                              grow.at[pl.ds(h, h)], sem).wait()

    mask_hi = jnp.int32(-65536)

    def compute(k, grow):
        def node(j, _):
            for t in range(OUT // 32):
                cs = pl.ds(t * 16, 16)
                w0 = grow[j * SEQ, cs]
                acc_lo = lax.bitcast_convert_type(w0 << 16, jnp.float32)
                acc_hi = lax.bitcast_convert_type(w0, jnp.float32)
                for s in range(1, SEQ):
                    w = grow[j * SEQ + s, cs]
                    acc_lo = acc_lo + lax.bitcast_convert_type(w << 16, jnp.float32)
                    acc_hi = acc_hi + lax.bitcast_convert_type(w, jnp.float32)
                acc_lo = acc_lo + bias_v[cs]
                acc_hi = acc_hi + bias_v[pl.ds(OUT // 2 + t * 16, 16)]
                elu_lo = jnp.where(acc_lo > 0.0, acc_lo, jnp.exp(acc_lo) - 1.0)
                elu_hi = jnp.where(acc_hi > 0.0, acc_hi, jnp.exp(acc_hi) - 1.0)
                ulo = lax.bitcast_convert_type(elu_lo, jnp.uint32)
                ulo = (ulo + jnp.uint32(0x7FFF) + ((ulo >> 16) & jnp.uint32(1))) >> 16
                uhi = lax.bitcast_convert_type(elu_hi, jnp.uint32)
                uhi = (uhi + jnp.uint32(0x7FFF) + ((uhi >> 16) & jnp.uint32(1))) & jnp.uint32(0xFFFF0000)
                outb_v[j, cs] = lax.bitcast_convert_type(ulo | uhi, jnp.int32)
            return 0
        lax.fori_loop(0, _GS_CH, node, 0)
        r0 = pl.multiple_of(row_base + k * _GS_CH, _GS_CH)
        pltpu.sync_copy(outb_v, out_hbm.at[pl.ds(r0, _GS_CH)])

    idx_start(0, idx8_a, sem_ia)
    build_start(0, idx8_a, sem_ia, gidx_a, grow_a, sem_a)
    idx_start(1, idx8_b, sem_ib)

    def pair(k2, _):
        k = 2 * k2
        build_start(k + 1, idx8_b, sem_ib, gidx_b, grow_b, sem_b)
        idx_start(k + 2, idx8_a, sem_ia)
        wait_for(gidx_a, grow_a, sem_a)
        compute(k, grow_a)

        pltpu.make_async_copy(idx_src(k + 2), idx8_a, sem_ia).wait()

        @pl.when(k + 2 < nch)
        def _():
            svec = iota16 * R + ((row_base + (k + 2) * _GS_CH) // N) * N
            for j in range(_GS_CH):
                gidx_a[pl.ds(j * SEQ, SEQ)] = idx8_a[j, :] + svec
            h = _GS_CH * SEQ // 2
            pltpu.async_copy(z4_hbm.at[gidx_a.at[pl.ds(0, h)]],
                             grow_a.at[pl.ds(0, h)], sem_a)
            pltpu.async_copy(z4_hbm.at[gidx_a.at[pl.ds(h, h)]],
                             grow_a.at[pl.ds(h, h)], sem_a)
        idx_start(k + 3, idx8_b, sem_ib)
        wait_for(gidx_b, grow_b, sem_b)
        compute(k + 1, grow_b)
        return 0

    lax.fori_loop(0, nch // 2, pair, 0)
    pltpu.make_async_copy(idx_src(0), idx8_b, sem_ib).wait()


def _gather_sum(z4, spiral_idx, bias):
    f = pl.kernel(
        _gather_sum_body,
        out_type=jax.ShapeDtypeStruct((R, OUT // 2), jnp.int32),
        mesh=_SC_MESH,
        scratch_types=[
            pltpu.VMEM((_GS_CH, SEQ), jnp.int32),
            pltpu.VMEM((_GS_CH, SEQ), jnp.int32),
            pltpu.VMEM((_GS_CH * SEQ,), jnp.int32),
            pltpu.VMEM((_GS_CH * SEQ,), jnp.int32),
            pltpu.VMEM((_GS_CH * SEQ, OUT // 2), jnp.int32),
            pltpu.VMEM((_GS_CH * SEQ, OUT // 2), jnp.int32),
            pltpu.VMEM((_GS_CH, OUT // 2), jnp.int32),
            pltpu.VMEM((OUT,), jnp.float32),
            pltpu.SemaphoreType.DMA,
            pltpu.SemaphoreType.DMA,
            pltpu.SemaphoreType.DMA,
            pltpu.SemaphoreType.DMA,
        ],
    )
    return f(z4, spiral_idx, bias)


NNZ_PAD = 20480
_P_C = 128
_NEB = NNZ_PAD // _P_C
M_PAD = 2504
_RPT = 312
_RPT_LAST = M_PAD - 7 * _RPT
_BCAST_DNUMS = lax.GatherDimensionNumbers(
    offset_dims=(), collapsed_slice_dims=(0,), start_index_map=(0,))


def _lane_bcast(v, j):
    return lax.gather(v, jnp.full((16, 1), j, jnp.int32), _BCAST_DNUMS,
                      (1,), mode=lax.GatherScatterMode.PROMISE_IN_BOUNDS)


def _pool_body(out2_hbm, ents_hbm, bounds_hbm, pooled_hbm,
               ent_a, gidx_a, gbuf_a, ent_b, gidx_b, gbuf_b,
               acc_v, bounds_v, sem_a, sem_b):
    cid = lax.axis_index("c")
    sid = lax.axis_index("s")
    wid = cid * 16 + sid
    b = 2 * cid + sid // 8
    t8 = sid % 8
    r_lo = t8 * _RPT
    iota16 = lax.broadcasted_iota(jnp.int32, (16,), 0)

    def zrow(i, _):
        for t in range(OUT // 16):
            acc_v[i, pl.ds(t * 16, 16)] = jnp.zeros((16,), jnp.float32)
        return 0
    lax.fori_loop(0, _RPT_LAST, zrow, 0)

    pltpu.sync_copy(bounds_hbm.at[pl.ds(wid * 16, 16)], bounds_v)
    bvec = bounds_v[pl.ds(0, 16)]
    ent_lo = bvec[0]
    ent_hi = bvec[1]
    cblk0 = ent_lo // _P_C
    nch = (ent_hi - cblk0 * _P_C + _P_C - 1) // _P_C
    mask_hi = jnp.int32(-65536)

    def fetch_start(c, ent_v, gidx_v, gbuf_v, sem):
        eoff = pl.multiple_of((cblk0 + c) * (3 * _P_C), 3 * _P_C)
        pltpu.sync_copy(ents_hbm.at[pl.ds(eoff, 3 * _P_C)], ent_v)
        for k in range(_P_C // 16):
            ks = pl.ds(k * 16, 16)
            gidx_v[ks] = ent_v[pl.ds(_P_C + k * 16, 16)] + b * N
        return pltpu.async_copy(out2_hbm.at[gidx_v], gbuf_v, sem)

    def process(c, ent_v, gidx_v, gbuf_v, sem):
        pltpu.make_async_copy(out2_hbm.at[gidx_v], gbuf_v, sem).wait()
        e_base = (cblk0 + c) * _P_C

        def group_body(g, _):
            gs = pl.ds(g * 16, 16)
            eid = e_base + g * 16 + iota16
            valid = (eid >= ent_lo) & (eid < ent_hi)
            vals16 = lax.bitcast_convert_type(
                ent_v[pl.ds(2 * _P_C + g * 16, 16)], jnp.float32)
            bcv = jnp.where(valid, vals16, 0.0)
            rowsv = ent_v[gs]
            for j in range(16):
                bc = _lane_bcast(bcv, j)
                lrow = jnp.clip(rowsv[j] - r_lo, 0, _RPT_LAST - 1)
                e = g * 16 + j
                for t in range(OUT // 32):
                    cs = pl.ds(t * 16, 16)
                    w = gbuf_v[e, cs]
                    lo = lax.bitcast_convert_type(w << 16, jnp.float32)
                    hi = lax.bitcast_convert_type(w & mask_hi, jnp.float32)
                    acc_v[lrow, cs] = acc_v[lrow, cs] + lo * bc
                    ch = pl.ds(OUT // 2 + t * 16, 16)
                    acc_v[lrow, ch] = acc_v[lrow, ch] + hi * bc
            return 0
        lax.fori_loop(0, _P_C // 16, group_body, 0)

    @pl.when(nch > 0)
    def _():
        fetch_start(0, ent_a, gidx_a, gbuf_a, sem_a)

    def pairs(k2, _):
        k = 2 * k2

        @pl.when(k + 1 < nch)
        def _():
            fetch_start(k + 1, ent_b, gidx_b, gbuf_b, sem_b)
        process(k, ent_a, gidx_a, gbuf_a, sem_a)

        @pl.when(k + 2 < nch)
        def _():
            fetch_start(k + 2, ent_a, gidx_a, gbuf_a, sem_a)

        @pl.when(k + 1 < nch)
        def _():
            process(k + 1, ent_b, gidx_b, gbuf_b, sem_b)
        return 0

    lax.fori_loop(0, (nch + 1) // 2, pairs, 0)

    @pl.when(t8 < 7)
    def _():
        pltpu.sync_copy(acc_v.at[pl.ds(0, _RPT)],
                        pooled_hbm.at[b].at[pl.ds(r_lo, _RPT)])

    @pl.when(t8 == 7)
    def _():
        pltpu.sync_copy(acc_v, pooled_hbm.at[b].at[pl.ds(7 * _RPT, _RPT_LAST)])


def _pool(out2, ents, bounds):
    f = pl.kernel(
        _pool_body,
        out_type=jax.ShapeDtypeStruct((B, M_PAD, OUT), jnp.float32),
        mesh=_SC_MESH,
        scratch_types=(
            2 * [
                pltpu.VMEM((3 * _P_C,), jnp.int32),
                pltpu.VMEM((_P_C,), jnp.int32),
                pltpu.VMEM((_P_C, OUT // 2), jnp.int32),
            ]
            + [
                pltpu.VMEM((_RPT_LAST, OUT), jnp.float32),
                pltpu.VMEM((16,), jnp.int32),
                pltpu.SemaphoreType.DMA,
                pltpu.SemaphoreType.DMA,
            ]
        ),
    )
    return f(out2, ents, bounds)


def kernel(x, spiral_indices, trans_row, trans_col, trans_val, W, b):
    x2 = x.reshape(R, IN)
    w2 = (W.astype(jnp.bfloat16)
          .reshape(SEQ // 2, 2, IN, OUT).transpose(0, 2, 1, 3)
          .reshape((SEQ // 2) * IN, 2 * OUT))
    z4 = _spiral_matmul(x2, w2).reshape(SEQ * R, OUT // 2)
    out2 = _gather_sum(z4, spiral_indices.astype(jnp.int32), b)

    pad = NNZ_PAD - NNZ
    rows32 = trans_row.astype(jnp.int32)
    rows_p = jnp.concatenate([rows32, jnp.zeros((pad,), jnp.int32)])
    cols_p = jnp.concatenate(
        [trans_col.astype(jnp.int32), jnp.zeros((pad,), jnp.int32)])
    vals_p = jnp.concatenate([trans_val, jnp.zeros((pad,), jnp.float32)])
    ents = jnp.concatenate(
        [rows_p.reshape(_NEB, _P_C), cols_p.reshape(_NEB, _P_C),
         lax.bitcast_convert_type(vals_p, jnp.int32).reshape(_NEB, _P_C)],
        axis=1).reshape(_NEB * 3 * _P_C)
    starts = jnp.arange(8, dtype=jnp.int32) * _RPT
    ss = jnp.searchsorted(rows32, starts, side="left").astype(jnp.int32)
    ends = jnp.concatenate([ss[1:], jnp.array([NNZ], jnp.int32)])
    pair16 = jnp.pad(jnp.stack([ss, ends], axis=1), ((0, 0), (0, 14)))
    bounds = jnp.tile(pair16, (4, 1)).reshape(32 * 16)
    return _pool(out2, ents, bounds)[:, :M, :]
